# baseline (speedup 1.0000x reference)
import functools

import jax
import jax.numpy as jnp
from jax import lax
from jax.experimental import pallas as pl
from jax.experimental.pallas import tpu as pltpu
from jax.experimental.pallas import tpu_sc as plsc

N = 10000
D = 128
E = 320000
NC = 2
NS = 16
NW = NC * NS
CH = 128
NCHUNK = 80
IB = 16
NB = NCHUNK // IB
EW = CH * NCHUNK
EPAD = EW * NW
NPAD = 10112
RPT = NPAD // NS
NBUF = 2
CA = 160
NBA = CA // IB
EA = NS * CA * CH


def _sc_deg_body(dst_hbm, ones_hbm, zeros_hbm, out_hbm, didx, ones, acc):
    c = lax.axis_index("c")
    s = lax.axis_index("s")
    w = c * NS + s
    r0 = s * RPT
    pltpu.sync_copy(dst_hbm.at[w], didx)
    pltpu.sync_copy(ones_hbm, ones)
    pltpu.sync_copy(zeros_hbm.at[pl.ds(r0, RPT)], acc.at[pl.ds(r0, RPT)])
    plsc.subcore_barrier()

    def body(j, carry):
        pltpu.sync_copy(ones, acc.at[didx.at[j]], add=True)
        return carry

    lax.fori_loop(0, NCHUNK, body, 0)
    plsc.subcore_barrier()
    pltpu.sync_copy(acc.at[pl.ds(r0, RPT)], out_hbm.at[c, pl.ds(r0, RPT), :])


def _agg_pipeline(hp_hbm, src_hbm, dst_hbm, s, nb, sidx, didx, rows, acc,
                  isem, sems):
    pltpu.sync_copy(src_hbm.at[s, pl.ds(0, IB)], sidx.at[0])
    pltpu.sync_copy(dst_hbm.at[s, pl.ds(0, IB)], didx.at[0])
    for h in range(nb):
        cur = h % 2
        nxt = (h + 1) % 2
        hn = (h + 1) % nb
        pltpu.async_copy(src_hbm.at[s, pl.ds(hn * IB, IB)], sidx.at[nxt], isem)
        pltpu.async_copy(dst_hbm.at[s, pl.ds(hn * IB, IB)], didx.at[nxt], isem)

        for b in range(NBUF):
            pltpu.async_copy(hp_hbm.at[sidx.at[cur, b]], rows.at[b], sems[b])

        def body(g, carry, cur=cur):
            for b in range(NBUF):
                j = g * NBUF + b
                pltpu.make_async_copy(hp_hbm.at[sidx.at[cur, j]], rows.at[b],
                                      sems[b]).wait()
                pltpu.sync_copy(rows.at[b], acc.at[didx.at[cur, j]], add=True)
                jn = lax.rem(j + NBUF, IB)
                pltpu.async_copy(hp_hbm.at[sidx.at[cur, jn]], rows.at[b],
                                 sems[b])
            return carry

        lax.fori_loop(0, IB // NBUF, body, 0)
        for b in range(NBUF):
            pltpu.make_async_copy(hp_hbm.at[sidx.at[cur, b]], rows.at[b],
                                  sems[b]).wait()
        pltpu.make_async_copy(src_hbm.at[s, pl.ds(hn * IB, IB)], sidx.at[nxt],
                              isem).wait()
        pltpu.make_async_copy(dst_hbm.at[s, pl.ds(hn * IB, IB)], didx.at[nxt],
                              isem).wait()


def _sc_agg_body(hp_hbm, src0_hbm, dst0_hbm, init_hbm, zeros_hbm, out_hbm,
                 sidx, didx, rows, acc, isem, s0, s1):
    sems = (s0, s1)
    c = lax.axis_index("c")
    s = lax.axis_index("s")
    r0 = s * RPT

    @pl.when(c == 0)
    def _():
        pltpu.sync_copy(init_hbm.at[pl.ds(r0, RPT)], acc.at[pl.ds(r0, RPT)])

    plsc.subcore_barrier()

    @pl.when(c == 0)
    def _():
        _agg_pipeline(hp_hbm, src0_hbm, dst0_hbm, s, NBA, sidx, didx, rows,
                      acc, isem, sems)

    plsc.subcore_barrier()

    @pl.when(c == 0)
    def _():
        pltpu.sync_copy(acc.at[pl.ds(r0, RPT)], out_hbm.at[pl.ds(r0, RPT), :])


@functools.cache
def _sc_kernels():
    mesh = plsc.VectorSubcoreMesh(core_axis_name="c", subcore_axis_name="s",
                                  num_cores=NC, num_subcores=NS)
    sc_deg = pl.kernel(
        _sc_deg_body,
        out_type=jax.ShapeDtypeStruct((NC, NPAD, D), jnp.float32),
        mesh=mesh,
        scratch_types=[
            pltpu.VMEM((NCHUNK, CH), jnp.int32),
            pltpu.VMEM((CH, D), jnp.float32),
            pltpu.VMEM_SHARED((NPAD, D), jnp.float32),
        ],
    )
    sc_agg = pl.kernel(
        _sc_agg_body,
        out_type=jax.ShapeDtypeStruct((NPAD, D), jnp.float32),
        mesh=mesh,
        scratch_types=[
            pltpu.VMEM((2, IB, CH), jnp.int32),
            pltpu.VMEM((2, IB, CH), jnp.int32),
            pltpu.VMEM((NBUF, CH, D), jnp.float32),
            pltpu.VMEM_SHARED((NPAD, D), jnp.float32),
            pltpu.SemaphoreType.DMA,
            pltpu.SemaphoreType.DMA,
            pltpu.SemaphoreType.DMA,
        ],
    )
    return sc_deg, sc_agg


def _dis_mask(deg_ref):
    deg = deg_ref[0, :, 0:1] + deg_ref[1, :, 0:1] + 1.0
    rows = lax.broadcasted_iota(jnp.int32, (NPAD, 1), 0)
    m = (rows < N).astype(jnp.float32)
    return m * lax.rsqrt(deg), m


def _tc_pre_body(deg_ref, x_ref, w_ref, hp_ref):
    dis, _ = _dis_mask(deg_ref)
    hp_ref[...] = dis * jnp.dot(x_ref[...], w_ref[...],
                                preferred_element_type=jnp.float32)


def _tc_mid_body(p_ref, deg_ref, b_ref, g_ref, beta_ref, w_ref, hp_ref):
    dis, m = _dis_mask(deg_ref)
    t = dis * p_ref[...] + m * b_ref[...]
    mean = jnp.sum(t, axis=0, keepdims=True) * (1.0 / N)
    var = jnp.sum(t * t, axis=0, keepdims=True) * (1.0 / N) - mean * mean
    y = (t - mean) * lax.rsqrt(var + 1e-5) * g_ref[...] + beta_ref[...]
    y = jnp.maximum(y, 0.0)
    hp_ref[...] = dis * jnp.dot(y, w_ref[...],
                                preferred_element_type=jnp.float32)


def _tc_fin_body(p_ref, deg_ref, b_ref, out_ref):
    dis, m = _dis_mask(deg_ref)
    out_ref[...] = dis * p_ref[...] + m * b_ref[...]


_hp_shape = jax.ShapeDtypeStruct((NPAD, D), jnp.float32)

_tc_pre = pl.pallas_call(_tc_pre_body, out_shape=_hp_shape)
_tc_mid = pl.pallas_call(_tc_mid_body, out_shape=_hp_shape)
_tc_fin = pl.pallas_call(
    _tc_fin_body, out_shape=jax.ShapeDtypeStruct((NPAD, D), jnp.float32))


def kernel(x, edge_index, W1, b1, g1, beta1, W2, b2, g2, beta2, W3, b3):
    src = edge_index[0]
    dst = edge_index[1]
    padv = jnp.full((EPAD - E,), N, dtype=jnp.int32)
    src_all = jnp.concatenate([src, padv])
    dst_all = jnp.concatenate([dst, padv])
    dst_p = dst_all.reshape(NW, NCHUNK, CH)
    src0 = src_all.reshape(NS, CA, CH)
    dst0 = dst_all.reshape(NS, CA, CH)
    x_p = jnp.pad(x, ((0, NPAD - N), (0, 0)))
    zeros_big = jnp.zeros((NPAD, D), jnp.float32)
    ones_rows = jnp.ones((CH, D), jnp.float32)
    b1r = b1.reshape(1, D)
    b2r = b2.reshape(1, D)
    b3r = b3.reshape(1, D)
    g1r = g1.reshape(1, D)
    g2r = g2.reshape(1, D)
    beta1r = beta1.reshape(1, D)
    beta2r = beta2.reshape(1, D)

    _sc_deg, _sc_agg = _sc_kernels()
    deg16 = _sc_deg(dst_p, ones_rows, zeros_big)
    hp1 = _tc_pre(deg16, x_p, W1)
    p1 = _sc_agg(hp1, src0, dst0, hp1, zeros_big)
    hp2 = _tc_mid(p1, deg16, b1r, g1r, beta1r, W2)
    p2 = _sc_agg(hp2, src0, dst0, hp2, zeros_big)
    hp3 = _tc_mid(p2, deg16, b2r, g2r, beta2r, W3)
    p3 = _sc_agg(hp3, src0, dst0, hp3, zeros_big)
    out = _tc_fin(p3, deg16, b3r)
    return out[:N]


# speedup vs baseline: 2.3550x; 2.3550x over previous
import functools

import jax
import jax.numpy as jnp
from jax import lax
from jax.experimental import pallas as pl
from jax.experimental.pallas import tpu as pltpu
from jax.experimental.pallas import tpu_sc as plsc

N = 10000
D = 128
E = 320000
NC = 2
NS = 16
NW = NC * NS
CH = 128
NCHUNK = 80
IB = 16
NB = NCHUNK // IB
EW = CH * NCHUNK
EPAD = EW * NW
NPAD = 10112
RPT = NPAD // NS
NBUF = 2
CA = 160
NBA = CA // IB
EA = NS * CA * CH


def _sc_deg_body(dst_hbm, ones_hbm, zeros_hbm, out_hbm, didx, ones, acc):
    c = lax.axis_index("c")
    s = lax.axis_index("s")
    w = c * NS + s
    r0 = s * RPT
    pltpu.sync_copy(dst_hbm.at[w], didx)
    pltpu.sync_copy(ones_hbm, ones)
    pltpu.sync_copy(zeros_hbm.at[pl.ds(r0, RPT)], acc.at[pl.ds(r0, RPT)])
    plsc.subcore_barrier()

    def body(j, carry):
        pltpu.sync_copy(ones, acc.at[didx.at[j]], add=True)
        return carry

    lax.fori_loop(0, NCHUNK, body, 0)
    plsc.subcore_barrier()
    pltpu.sync_copy(acc.at[pl.ds(r0, RPT)], out_hbm.at[c, pl.ds(r0, RPT), :])


def _agg_pipeline(hp_hbm, src_hbm, dst_hbm, s, nb, sidx, didx, rows, acc,
                  isem, sems):
    pltpu.sync_copy(src_hbm.at[s, pl.ds(0, IB)], sidx.at[0])
    pltpu.sync_copy(dst_hbm.at[s, pl.ds(0, IB)], didx.at[0])
    for h in range(nb):
        cur = h % 2
        nxt = (h + 1) % 2
        hn = (h + 1) % nb
        pltpu.async_copy(src_hbm.at[s, pl.ds(hn * IB, IB)], sidx.at[nxt], isem)
        pltpu.async_copy(dst_hbm.at[s, pl.ds(hn * IB, IB)], didx.at[nxt], isem)

        for b in range(NBUF):
            pltpu.async_copy(hp_hbm.at[sidx.at[cur, b]], rows.at[b], sems[b])

        def body(g, carry, cur=cur):
            for b in range(NBUF):
                j = g * NBUF + b
                pltpu.make_async_copy(hp_hbm.at[sidx.at[cur, j]], rows.at[b],
                                      sems[b]).wait()
                pltpu.sync_copy(rows.at[b], acc.at[didx.at[cur, j]], add=True)
                jn = lax.rem(j + NBUF, IB)
                pltpu.async_copy(hp_hbm.at[sidx.at[cur, jn]], rows.at[b],
                                 sems[b])
            return carry

        lax.fori_loop(0, IB // NBUF, body, 0)
        for b in range(NBUF):
            pltpu.make_async_copy(hp_hbm.at[sidx.at[cur, b]], rows.at[b],
                                  sems[b]).wait()
        pltpu.make_async_copy(src_hbm.at[s, pl.ds(hn * IB, IB)], sidx.at[nxt],
                              isem).wait()
        pltpu.make_async_copy(dst_hbm.at[s, pl.ds(hn * IB, IB)], didx.at[nxt],
                              isem).wait()


def _sc_agg_body(hp_hbm, src0_hbm, dst0_hbm, init_hbm, zeros_hbm, out_hbm,
                 sidx, didx, rows, acc, isem, s0, s1):
    sems = (s0, s1)
    c = lax.axis_index("c")
    s = lax.axis_index("s")
    r0 = s * RPT

    @pl.when(c == 0)
    def _():
        pltpu.sync_copy(init_hbm.at[pl.ds(r0, RPT)], acc.at[pl.ds(r0, RPT)])

    plsc.subcore_barrier()

    @pl.when(c == 0)
    def _():
        _agg_pipeline(hp_hbm, src0_hbm, dst0_hbm, s, NBA, sidx, didx, rows,
                      acc, isem, sems)

    plsc.subcore_barrier()

    @pl.when(c == 0)
    def _():
        pltpu.sync_copy(acc.at[pl.ds(r0, RPT)], out_hbm.at[pl.ds(r0, RPT), :])


@functools.cache
def _sc_kernels():
    mesh = plsc.VectorSubcoreMesh(core_axis_name="c", subcore_axis_name="s",
                                  num_cores=NC, num_subcores=NS)
    sc_deg = pl.kernel(
        _sc_deg_body,
        out_type=jax.ShapeDtypeStruct((NC, NPAD, D), jnp.float32),
        mesh=mesh,
        scratch_types=[
            pltpu.VMEM((NCHUNK, CH), jnp.int32),
            pltpu.VMEM((CH, D), jnp.float32),
            pltpu.VMEM_SHARED((NPAD, D), jnp.float32),
        ],
    )
    sc_agg = pl.kernel(
        _sc_agg_body,
        out_type=jax.ShapeDtypeStruct((NPAD, D), jnp.float32),
        mesh=mesh,
        scratch_types=[
            pltpu.VMEM((2, IB, CH), jnp.int32),
            pltpu.VMEM((2, IB, CH), jnp.int32),
            pltpu.VMEM((NBUF, CH, D), jnp.float32),
            pltpu.VMEM_SHARED((NPAD, D), jnp.float32),
            pltpu.SemaphoreType.DMA,
            pltpu.SemaphoreType.DMA,
            pltpu.SemaphoreType.DMA,
        ],
    )
    return sc_deg, sc_agg


def _dis_mask(deg_ref):
    deg = deg_ref[0, :, 0:1] + deg_ref[1, :, 0:1] + 1.0
    rows = lax.broadcasted_iota(jnp.int32, (NPAD, 1), 0)
    m = (rows < N).astype(jnp.float32)
    return m * lax.rsqrt(deg), m


def _tc_pre_body(deg_ref, x_ref, w_ref, hp_ref):
    dis, _ = _dis_mask(deg_ref)
    hp_ref[...] = dis * jnp.dot(x_ref[...], w_ref[...],
                                preferred_element_type=jnp.float32)


def _tc_mid_body(p_ref, deg_ref, b_ref, g_ref, beta_ref, w_ref, hp_ref):
    dis, m = _dis_mask(deg_ref)
    t = dis * p_ref[...] + m * b_ref[...]
    mean = jnp.sum(t, axis=0, keepdims=True) * (1.0 / N)
    var = jnp.sum(t * t, axis=0, keepdims=True) * (1.0 / N) - mean * mean
    y = (t - mean) * lax.rsqrt(var + 1e-5) * g_ref[...] + beta_ref[...]
    y = jnp.maximum(y, 0.0)
    hp_ref[...] = dis * jnp.dot(y, w_ref[...],
                                preferred_element_type=jnp.float32)


def _tc_fin_body(p_ref, deg_ref, b_ref, out_ref):
    dis, m = _dis_mask(deg_ref)
    out_ref[...] = dis * p_ref[...] + m * b_ref[...]


_hp_shape = jax.ShapeDtypeStruct((NPAD, D), jnp.float32)

_tc_pre = pl.pallas_call(_tc_pre_body, out_shape=_hp_shape)
_tc_mid = pl.pallas_call(_tc_mid_body, out_shape=_hp_shape)
_tc_fin = pl.pallas_call(
    _tc_fin_body, out_shape=jax.ShapeDtypeStruct((NPAD, D), jnp.float32))


def kernel(x, edge_index, W1, b1, g1, beta1, W2, b2, g2, beta2, W3, b3):
    src = edge_index[0]
    dst = edge_index[1]
    padv = N + (jnp.arange(EPAD - E, dtype=jnp.int32) % (NPAD - N))
    src_all = jnp.concatenate([src, padv])
    dst_all = jnp.concatenate([dst, padv])
    dst_p = dst_all.reshape(NW, NCHUNK, CH)
    src0 = src_all.reshape(NS, CA, CH)
    dst0 = dst_all.reshape(NS, CA, CH)
    x_p = jnp.pad(x, ((0, NPAD - N), (0, 0)))
    zeros_big = jnp.zeros((NPAD, D), jnp.float32)
    ones_rows = jnp.ones((CH, D), jnp.float32)
    b1r = b1.reshape(1, D)
    b2r = b2.reshape(1, D)
    b3r = b3.reshape(1, D)
    g1r = g1.reshape(1, D)
    g2r = g2.reshape(1, D)
    beta1r = beta1.reshape(1, D)
    beta2r = beta2.reshape(1, D)

    _sc_deg, _sc_agg = _sc_kernels()
    deg16 = _sc_deg(dst_p, ones_rows, zeros_big)
    hp1 = _tc_pre(deg16, x_p, W1)
    p1 = _sc_agg(hp1, src0, dst0, hp1, zeros_big)
    hp2 = _tc_mid(p1, deg16, b1r, g1r, beta1r, W2)
    p2 = _sc_agg(hp2, src0, dst0, hp2, zeros_big)
    hp3 = _tc_mid(p2, deg16, b2r, g2r, beta2r, W3)
    p3 = _sc_agg(hp3, src0, dst0, hp3, zeros_big)
    out = _tc_fin(p3, deg16, b3r)
    return out[:N]


# speedup vs baseline: 2.7258x; 1.1574x over previous
import functools

import jax
import jax.numpy as jnp
from jax import lax
from jax.experimental import pallas as pl
from jax.experimental.pallas import tpu as pltpu
from jax.experimental.pallas import tpu_sc as plsc

N = 10000
D = 128
E = 320000
NC = 2
NS = 16
NW = NC * NS
CH = 128
NCHUNK = 80
IB = 16
NB = NCHUNK // IB
EW = CH * NCHUNK
EPAD = EW * NW
NPAD = 10112
RPT = NPAD // NS
NBUF = 2
CA = 128
CB = 32
NBA = CA // IB
NBB = CB // IB
EA = NS * CA * CH
EB = NS * CB * CH


def _sc_deg_body(dst_hbm, ones_hbm, zeros_hbm, out_hbm, didx, ones, acc):
    c = lax.axis_index("c")
    s = lax.axis_index("s")
    w = c * NS + s
    r0 = s * RPT
    pltpu.sync_copy(dst_hbm.at[w], didx)
    pltpu.sync_copy(ones_hbm, ones)
    pltpu.sync_copy(zeros_hbm.at[pl.ds(r0, RPT)], acc.at[pl.ds(r0, RPT)])
    plsc.subcore_barrier()

    def body(j, carry):
        pltpu.sync_copy(ones, acc.at[didx.at[j]], add=True)
        return carry

    lax.fori_loop(0, NCHUNK, body, 0)
    plsc.subcore_barrier()
    pltpu.sync_copy(acc.at[pl.ds(r0, RPT)], out_hbm.at[c, pl.ds(r0, RPT), :])


def _agg_pipeline(hp_hbm, src_hbm, dst_hbm, s, nb, sidx, didx, rows, acc,
                  isem, sems):
    pltpu.sync_copy(src_hbm.at[s, pl.ds(0, IB)], sidx.at[0])
    pltpu.sync_copy(dst_hbm.at[s, pl.ds(0, IB)], didx.at[0])
    for h in range(nb):
        cur = h % 2
        nxt = (h + 1) % 2
        hn = (h + 1) % nb
        pltpu.async_copy(src_hbm.at[s, pl.ds(hn * IB, IB)], sidx.at[nxt], isem)
        pltpu.async_copy(dst_hbm.at[s, pl.ds(hn * IB, IB)], didx.at[nxt], isem)

        for b in range(NBUF):
            pltpu.async_copy(hp_hbm.at[sidx.at[cur, b]], rows.at[b], sems[b])

        def body(g, carry, cur=cur):
            for b in range(NBUF):
                j = g * NBUF + b
                pltpu.make_async_copy(hp_hbm.at[sidx.at[cur, j]], rows.at[b],
                                      sems[b]).wait()
                pltpu.sync_copy(rows.at[b], acc.at[didx.at[cur, j]], add=True)
                jn = lax.rem(j + NBUF, IB)
                pltpu.async_copy(hp_hbm.at[sidx.at[cur, jn]], rows.at[b],
                                 sems[b])
            return carry

        lax.fori_loop(0, IB // NBUF, body, 0)
        for b in range(NBUF):
            pltpu.make_async_copy(hp_hbm.at[sidx.at[cur, b]], rows.at[b],
                                  sems[b]).wait()
        pltpu.make_async_copy(src_hbm.at[s, pl.ds(hn * IB, IB)], sidx.at[nxt],
                              isem).wait()
        pltpu.make_async_copy(dst_hbm.at[s, pl.ds(hn * IB, IB)], didx.at[nxt],
                              isem).wait()


def _sc_agg_body(hp_hbm, src0_hbm, dst0_hbm, src1_hbm, dst1_hbm, init_hbm,
                 zeros_hbm, out_hbm, sidx, didx, rows, acc, isem, s0, s1):
    sems = (s0, s1)
    c = lax.axis_index("c")
    s = lax.axis_index("s")
    r0 = s * RPT

    @pl.when(c == 0)
    def _():
        pltpu.sync_copy(init_hbm.at[pl.ds(r0, RPT)], acc.at[pl.ds(r0, RPT)])

    @pl.when(c != 0)
    def _():
        pltpu.sync_copy(zeros_hbm.at[pl.ds(r0, RPT)], acc.at[pl.ds(r0, RPT)])

    plsc.subcore_barrier()

    @pl.when(c == 0)
    def _():
        _agg_pipeline(hp_hbm, src0_hbm, dst0_hbm, s, NBA, sidx, didx, rows,
                      acc, isem, sems)

    @pl.when(c != 0)
    def _():
        _agg_pipeline(hp_hbm, src1_hbm, dst1_hbm, s, NBB, sidx, didx, rows,
                      acc, isem, sems)

    plsc.subcore_barrier()
    pltpu.sync_copy(acc.at[pl.ds(r0, RPT)], out_hbm.at[c, pl.ds(r0, RPT), :])


@functools.cache
def _sc_kernels():
    mesh = plsc.VectorSubcoreMesh(core_axis_name="c", subcore_axis_name="s",
                                  num_cores=NC, num_subcores=NS)
    sc_deg = pl.kernel(
        _sc_deg_body,
        out_type=jax.ShapeDtypeStruct((NC, NPAD, D), jnp.float32),
        mesh=mesh,
        scratch_types=[
            pltpu.VMEM((NCHUNK, CH), jnp.int32),
            pltpu.VMEM((CH, D), jnp.float32),
            pltpu.VMEM_SHARED((NPAD, D), jnp.float32),
        ],
    )
    sc_agg = pl.kernel(
        _sc_agg_body,
        out_type=jax.ShapeDtypeStruct((NC, NPAD, D), jnp.float32),
        mesh=mesh,
        scratch_types=[
            pltpu.VMEM((2, IB, CH), jnp.int32),
            pltpu.VMEM((2, IB, CH), jnp.int32),
            pltpu.VMEM((NBUF, CH, D), jnp.float32),
            pltpu.VMEM_SHARED((NPAD, D), jnp.float32),
            pltpu.SemaphoreType.DMA,
            pltpu.SemaphoreType.DMA,
            pltpu.SemaphoreType.DMA,
        ],
    )
    return sc_deg, sc_agg


def _dis_mask(deg_ref):
    deg = deg_ref[0, :, 0:1] + deg_ref[1, :, 0:1] + 1.0
    rows = lax.broadcasted_iota(jnp.int32, (NPAD, 1), 0)
    m = (rows < N).astype(jnp.float32)
    return m * lax.rsqrt(deg), m


def _tc_pre_body(deg_ref, x_ref, w_ref, hp_ref):
    dis, _ = _dis_mask(deg_ref)
    hp_ref[...] = dis * jnp.dot(x_ref[...], w_ref[...],
                                preferred_element_type=jnp.float32)


def _tc_mid_body(p_ref, deg_ref, b_ref, g_ref, beta_ref, w_ref, hp_ref):
    dis, m = _dis_mask(deg_ref)
    t = dis * (p_ref[0] + p_ref[1]) + m * b_ref[...]
    mean = jnp.sum(t, axis=0, keepdims=True) * (1.0 / N)
    var = jnp.sum(t * t, axis=0, keepdims=True) * (1.0 / N) - mean * mean
    y = (t - mean) * lax.rsqrt(var + 1e-5) * g_ref[...] + beta_ref[...]
    y = jnp.maximum(y, 0.0)
    hp_ref[...] = dis * jnp.dot(y, w_ref[...],
                                preferred_element_type=jnp.float32)


def _tc_fin_body(p_ref, deg_ref, b_ref, out_ref):
    dis, m = _dis_mask(deg_ref)
    out_ref[...] = dis * (p_ref[0] + p_ref[1]) + m * b_ref[...]


_hp_shape = jax.ShapeDtypeStruct((NPAD, D), jnp.float32)

_tc_pre = pl.pallas_call(_tc_pre_body, out_shape=_hp_shape)
_tc_mid = pl.pallas_call(_tc_mid_body, out_shape=_hp_shape)
_tc_fin = pl.pallas_call(
    _tc_fin_body, out_shape=jax.ShapeDtypeStruct((NPAD, D), jnp.float32))


def kernel(x, edge_index, W1, b1, g1, beta1, W2, b2, g2, beta2, W3, b3):
    src = edge_index[0]
    dst = edge_index[1]
    padv = N + (jnp.arange(EPAD - E, dtype=jnp.int32) % (NPAD - N))
    src_all = jnp.concatenate([src, padv])
    dst_all = jnp.concatenate([dst, padv])
    dst_p = dst_all.reshape(NW, NCHUNK, CH)
    src0 = src_all[:EA].reshape(NS, CA, CH)
    dst0 = dst_all[:EA].reshape(NS, CA, CH)
    src1 = src_all[EA:].reshape(NS, CB, CH)
    dst1 = dst_all[EA:].reshape(NS, CB, CH)
    x_p = jnp.pad(x, ((0, NPAD - N), (0, 0)))
    zeros_big = jnp.zeros((NPAD, D), jnp.float32)
    ones_rows = jnp.ones((CH, D), jnp.float32)
    b1r = b1.reshape(1, D)
    b2r = b2.reshape(1, D)
    b3r = b3.reshape(1, D)
    g1r = g1.reshape(1, D)
    g2r = g2.reshape(1, D)
    beta1r = beta1.reshape(1, D)
    beta2r = beta2.reshape(1, D)

    _sc_deg, _sc_agg = _sc_kernels()
    deg16 = _sc_deg(dst_p, ones_rows, zeros_big)
    hp1 = _tc_pre(deg16, x_p, W1)
    p1 = _sc_agg(hp1, src0, dst0, src1, dst1, hp1, zeros_big)
    hp2 = _tc_mid(p1, deg16, b1r, g1r, beta1r, W2)
    p2 = _sc_agg(hp2, src0, dst0, src1, dst1, hp2, zeros_big)
    hp3 = _tc_mid(p2, deg16, b2r, g2r, beta2r, W3)
    p3 = _sc_agg(hp3, src0, dst0, src1, dst1, hp3, zeros_big)
    out = _tc_fin(p3, deg16, b3r)
    return out[:N]


# speedup vs baseline: 3.2108x; 1.1779x over previous
import functools

import jax
import jax.numpy as jnp
from jax import lax
from jax.experimental import pallas as pl
from jax.experimental.pallas import tpu as pltpu
from jax.experimental.pallas import tpu_sc as plsc

N = 10000
D = 128
E = 320000
NC = 2
NS = 16
NW = NC * NS
CH = 128
NCHUNK = 80
IB = 8
NB = NCHUNK // IB
EW = CH * NCHUNK
EPAD = EW * NW
NPAD = 10112
RPT = NPAD // NS
NBUF = 2
CA = 88
CB = 72
NBA = CA // IB
NBB = CB // IB
EA = NS * CA * CH
EB = NS * CB * CH


def _sc_deg_body(dst_hbm, ones_hbm, zeros_hbm, out_hbm, didx, ones, acc):
    c = lax.axis_index("c")
    s = lax.axis_index("s")
    w = c * NS + s
    r0 = s * RPT
    pltpu.sync_copy(dst_hbm.at[w], didx)
    pltpu.sync_copy(ones_hbm, ones)
    pltpu.sync_copy(zeros_hbm.at[pl.ds(r0, RPT)], acc.at[pl.ds(r0, RPT)])
    plsc.subcore_barrier()

    def body(j, carry):
        pltpu.sync_copy(ones, acc.at[didx.at[j]], add=True)
        return carry

    lax.fori_loop(0, NCHUNK, body, 0)
    plsc.subcore_barrier()
    pltpu.sync_copy(acc.at[pl.ds(r0, RPT)], out_hbm.at[c, pl.ds(r0, RPT), :])


def _agg_pipeline(hp_hbm, src_hbm, dst_hbm, s, nb, sidx, didx, rows, acc,
                  isem, sems):
    pltpu.sync_copy(src_hbm.at[s, pl.ds(0, IB)], sidx.at[0])
    pltpu.sync_copy(dst_hbm.at[s, pl.ds(0, IB)], didx.at[0])
    for h in range(nb):
        cur = h % 2
        nxt = (h + 1) % 2
        hn = (h + 1) % nb
        pltpu.async_copy(src_hbm.at[s, pl.ds(hn * IB, IB)], sidx.at[nxt], isem)
        pltpu.async_copy(dst_hbm.at[s, pl.ds(hn * IB, IB)], didx.at[nxt], isem)

        for b in range(NBUF):
            pltpu.async_copy(hp_hbm.at[sidx.at[cur, b]], rows.at[b], sems[b])

        def body(g, carry, cur=cur):
            for b in range(NBUF):
                j = g * NBUF + b
                pltpu.make_async_copy(hp_hbm.at[sidx.at[cur, j]], rows.at[b],
                                      sems[b]).wait()
                pltpu.sync_copy(rows.at[b], acc.at[didx.at[cur, j]], add=True)
                jn = lax.rem(j + NBUF, IB)
                pltpu.async_copy(hp_hbm.at[sidx.at[cur, jn]], rows.at[b],
                                 sems[b])
            return carry

        lax.fori_loop(0, IB // NBUF, body, 0)
        for b in range(NBUF):
            pltpu.make_async_copy(hp_hbm.at[sidx.at[cur, b]], rows.at[b],
                                  sems[b]).wait()
        pltpu.make_async_copy(src_hbm.at[s, pl.ds(hn * IB, IB)], sidx.at[nxt],
                              isem).wait()
        pltpu.make_async_copy(dst_hbm.at[s, pl.ds(hn * IB, IB)], didx.at[nxt],
                              isem).wait()


def _sc_agg_body(hp_hbm, src0_hbm, dst0_hbm, src1_hbm, dst1_hbm, init_hbm,
                 zeros_hbm, out_hbm, sidx, didx, rows, acc, isem, s0, s1):
    sems = (s0, s1)
    c = lax.axis_index("c")
    s = lax.axis_index("s")
    r0 = s * RPT

    @pl.when(c == 0)
    def _():
        pltpu.sync_copy(init_hbm.at[pl.ds(r0, RPT)], acc.at[pl.ds(r0, RPT)])

    @pl.when(c != 0)
    def _():
        pltpu.sync_copy(zeros_hbm.at[pl.ds(r0, RPT)], acc.at[pl.ds(r0, RPT)])

    plsc.subcore_barrier()

    @pl.when(c == 0)
    def _():
        _agg_pipeline(hp_hbm, src0_hbm, dst0_hbm, s, NBA, sidx, didx, rows,
                      acc, isem, sems)

    @pl.when(c != 0)
    def _():
        _agg_pipeline(hp_hbm, src1_hbm, dst1_hbm, s, NBB, sidx, didx, rows,
                      acc, isem, sems)

    plsc.subcore_barrier()
    pltpu.sync_copy(acc.at[pl.ds(r0, RPT)], out_hbm.at[c, pl.ds(r0, RPT), :])


@functools.cache
def _sc_kernels():
    mesh = plsc.VectorSubcoreMesh(core_axis_name="c", subcore_axis_name="s",
                                  num_cores=NC, num_subcores=NS)
    sc_deg = pl.kernel(
        _sc_deg_body,
        out_type=jax.ShapeDtypeStruct((NC, NPAD, D), jnp.float32),
        mesh=mesh,
        scratch_types=[
            pltpu.VMEM((NCHUNK, CH), jnp.int32),
            pltpu.VMEM((CH, D), jnp.float32),
            pltpu.VMEM_SHARED((NPAD, D), jnp.float32),
        ],
    )
    sc_agg = pl.kernel(
        _sc_agg_body,
        out_type=jax.ShapeDtypeStruct((NC, NPAD, D), jnp.float32),
        mesh=mesh,
        scratch_types=[
            pltpu.VMEM((2, IB, CH), jnp.int32),
            pltpu.VMEM((2, IB, CH), jnp.int32),
            pltpu.VMEM((NBUF, CH, D), jnp.float32),
            pltpu.VMEM_SHARED((NPAD, D), jnp.float32),
            pltpu.SemaphoreType.DMA,
            pltpu.SemaphoreType.DMA,
            pltpu.SemaphoreType.DMA,
        ],
    )
    return sc_deg, sc_agg


def _dis_mask(deg_ref):
    deg = deg_ref[0, :, 0:1] + deg_ref[1, :, 0:1] + 1.0
    rows = lax.broadcasted_iota(jnp.int32, (NPAD, 1), 0)
    m = (rows < N).astype(jnp.float32)
    return m * lax.rsqrt(deg), m


def _tc_pre_body(deg_ref, x_ref, w_ref, hp_ref):
    dis, _ = _dis_mask(deg_ref)
    hp_ref[...] = dis * jnp.dot(x_ref[...], w_ref[...],
                                preferred_element_type=jnp.float32)


def _tc_mid_body(p_ref, deg_ref, b_ref, g_ref, beta_ref, w_ref, hp_ref):
    dis, m = _dis_mask(deg_ref)
    t = dis * (p_ref[0] + p_ref[1]) + m * b_ref[...]
    mean = jnp.sum(t, axis=0, keepdims=True) * (1.0 / N)
    var = jnp.sum(t * t, axis=0, keepdims=True) * (1.0 / N) - mean * mean
    y = (t - mean) * lax.rsqrt(var + 1e-5) * g_ref[...] + beta_ref[...]
    y = jnp.maximum(y, 0.0)
    hp_ref[...] = dis * jnp.dot(y, w_ref[...],
                                preferred_element_type=jnp.float32)


def _tc_fin_body(p_ref, deg_ref, b_ref, out_ref):
    dis, m = _dis_mask(deg_ref)
    out_ref[...] = dis * (p_ref[0] + p_ref[1]) + m * b_ref[...]


_hp_shape = jax.ShapeDtypeStruct((NPAD, D), jnp.float32)

_tc_pre = pl.pallas_call(_tc_pre_body, out_shape=_hp_shape)
_tc_mid = pl.pallas_call(_tc_mid_body, out_shape=_hp_shape)
_tc_fin = pl.pallas_call(
    _tc_fin_body, out_shape=jax.ShapeDtypeStruct((NPAD, D), jnp.float32))


def kernel(x, edge_index, W1, b1, g1, beta1, W2, b2, g2, beta2, W3, b3):
    src = edge_index[0]
    dst = edge_index[1]
    padv = N + (jnp.arange(EPAD - E, dtype=jnp.int32) % (NPAD - N))
    src_all = jnp.concatenate([src, padv])
    dst_all = jnp.concatenate([dst, padv])
    dst_p = dst_all.reshape(NW, NCHUNK, CH)
    src0 = src_all[:EA].reshape(NS, CA, CH)
    dst0 = dst_all[:EA].reshape(NS, CA, CH)
    src1 = src_all[EA:].reshape(NS, CB, CH)
    dst1 = dst_all[EA:].reshape(NS, CB, CH)
    x_p = jnp.pad(x, ((0, NPAD - N), (0, 0)))
    zeros_big = jnp.zeros((NPAD, D), jnp.float32)
    ones_rows = jnp.ones((CH, D), jnp.float32)
    b1r = b1.reshape(1, D)
    b2r = b2.reshape(1, D)
    b3r = b3.reshape(1, D)
    g1r = g1.reshape(1, D)
    g2r = g2.reshape(1, D)
    beta1r = beta1.reshape(1, D)
    beta2r = beta2.reshape(1, D)

    _sc_deg, _sc_agg = _sc_kernels()
    deg16 = _sc_deg(dst_p, ones_rows, zeros_big)
    hp1 = _tc_pre(deg16, x_p, W1)
    p1 = _sc_agg(hp1, src0, dst0, src1, dst1, hp1, zeros_big)
    hp2 = _tc_mid(p1, deg16, b1r, g1r, beta1r, W2)
    p2 = _sc_agg(hp2, src0, dst0, src1, dst1, hp2, zeros_big)
    hp3 = _tc_mid(p2, deg16, b2r, g2r, beta2r, W3)
    p3 = _sc_agg(hp3, src0, dst0, src1, dst1, hp3, zeros_big)
    out = _tc_fin(p3, deg16, b3r)
    return out[:N]


# speedup vs baseline: 3.3718x; 1.0501x over previous
import functools

import jax
import jax.numpy as jnp
from jax import lax
from jax.experimental import pallas as pl
from jax.experimental.pallas import tpu as pltpu
from jax.experimental.pallas import tpu_sc as plsc

N = 10000
D = 128
E = 320000
NC = 2
NS = 16
NW = NC * NS
CH = 128
NCHUNK = 80
IB = 8
NB = NCHUNK // IB
EW = CH * NCHUNK
EPAD = EW * NW
NPAD = 10112
RPT = NPAD // NS
NBUF = 2
CA = 80
CB = 80
NBA = CA // IB
NBB = CB // IB
EA = NS * CA * CH
EB = NS * CB * CH


def _sc_deg_body(dst_hbm, ones_hbm, zeros_hbm, out_hbm, didx, ones, acc):
    c = lax.axis_index("c")
    s = lax.axis_index("s")
    w = c * NS + s
    r0 = s * RPT
    pltpu.sync_copy(dst_hbm.at[w], didx)
    pltpu.sync_copy(ones_hbm, ones)
    pltpu.sync_copy(zeros_hbm.at[pl.ds(r0, RPT)], acc.at[pl.ds(r0, RPT)])
    plsc.subcore_barrier()

    def body(j, carry):
        pltpu.sync_copy(ones, acc.at[didx.at[j]], add=True)
        return carry

    lax.fori_loop(0, NCHUNK, body, 0)
    plsc.subcore_barrier()
    pltpu.sync_copy(acc.at[pl.ds(r0, RPT)], out_hbm.at[c, pl.ds(r0, RPT), :])


def _agg_pipeline(hp_hbm, src_hbm, dst_hbm, s, nb, sidx, didx, rows, acc,
                  isem, sems):
    pltpu.sync_copy(src_hbm.at[s, pl.ds(0, IB)], sidx.at[0])
    pltpu.sync_copy(dst_hbm.at[s, pl.ds(0, IB)], didx.at[0])
    for h in range(nb):
        cur = h % 2
        nxt = (h + 1) % 2
        hn = (h + 1) % nb
        pltpu.async_copy(src_hbm.at[s, pl.ds(hn * IB, IB)], sidx.at[nxt], isem)
        pltpu.async_copy(dst_hbm.at[s, pl.ds(hn * IB, IB)], didx.at[nxt], isem)

        for b in range(NBUF):
            pltpu.async_copy(hp_hbm.at[sidx.at[cur, b]], rows.at[b], sems[b])

        def body(g, carry, cur=cur):
            for b in range(NBUF):
                j = g * NBUF + b
                pltpu.make_async_copy(hp_hbm.at[sidx.at[cur, j]], rows.at[b],
                                      sems[b]).wait()
                pltpu.sync_copy(rows.at[b], acc.at[didx.at[cur, j]], add=True)
                jn = lax.rem(j + NBUF, IB)
                pltpu.async_copy(hp_hbm.at[sidx.at[cur, jn]], rows.at[b],
                                 sems[b])
            return carry

        lax.fori_loop(0, IB // NBUF, body, 0)
        for b in range(NBUF):
            pltpu.make_async_copy(hp_hbm.at[sidx.at[cur, b]], rows.at[b],
                                  sems[b]).wait()
        pltpu.make_async_copy(src_hbm.at[s, pl.ds(hn * IB, IB)], sidx.at[nxt],
                              isem).wait()
        pltpu.make_async_copy(dst_hbm.at[s, pl.ds(hn * IB, IB)], didx.at[nxt],
                              isem).wait()


def _sc_agg_body(hp_hbm, src0_hbm, dst0_hbm, src1_hbm, dst1_hbm, init_hbm,
                 zeros_hbm, out_hbm, sidx, didx, rows, acc, isem, s0, s1):
    sems = (s0, s1)
    c = lax.axis_index("c")
    s = lax.axis_index("s")
    r0 = s * RPT

    @pl.when(c == 0)
    def _():
        pltpu.sync_copy(init_hbm.at[pl.ds(r0, RPT)], acc.at[pl.ds(r0, RPT)])

    @pl.when(c != 0)
    def _():
        pltpu.sync_copy(zeros_hbm.at[pl.ds(r0, RPT)], acc.at[pl.ds(r0, RPT)])

    plsc.subcore_barrier()

    @pl.when(c == 0)
    def _():
        _agg_pipeline(hp_hbm, src0_hbm, dst0_hbm, s, NBA, sidx, didx, rows,
                      acc, isem, sems)

    @pl.when(c != 0)
    def _():
        _agg_pipeline(hp_hbm, src1_hbm, dst1_hbm, s, NBB, sidx, didx, rows,
                      acc, isem, sems)

    plsc.subcore_barrier()
    pltpu.sync_copy(acc.at[pl.ds(r0, RPT)], out_hbm.at[c, pl.ds(r0, RPT), :])


@functools.cache
def _sc_kernels():
    mesh = plsc.VectorSubcoreMesh(core_axis_name="c", subcore_axis_name="s",
                                  num_cores=NC, num_subcores=NS)
    sc_deg = pl.kernel(
        _sc_deg_body,
        out_type=jax.ShapeDtypeStruct((NC, NPAD, D), jnp.float32),
        mesh=mesh,
        scratch_types=[
            pltpu.VMEM((NCHUNK, CH), jnp.int32),
            pltpu.VMEM((CH, D), jnp.float32),
            pltpu.VMEM_SHARED((NPAD, D), jnp.float32),
        ],
    )
    sc_agg = pl.kernel(
        _sc_agg_body,
        out_type=jax.ShapeDtypeStruct((NC, NPAD, D), jnp.float32),
        mesh=mesh,
        scratch_types=[
            pltpu.VMEM((2, IB, CH), jnp.int32),
            pltpu.VMEM((2, IB, CH), jnp.int32),
            pltpu.VMEM((NBUF, CH, D), jnp.float32),
            pltpu.VMEM_SHARED((NPAD, D), jnp.float32),
            pltpu.SemaphoreType.DMA,
            pltpu.SemaphoreType.DMA,
            pltpu.SemaphoreType.DMA,
        ],
    )
    return sc_deg, sc_agg


def _dis_mask(deg_ref):
    deg = deg_ref[0, :, 0:1] + deg_ref[1, :, 0:1] + 1.0
    rows = lax.broadcasted_iota(jnp.int32, (NPAD, 1), 0)
    m = (rows < N).astype(jnp.float32)
    return m * lax.rsqrt(deg), m


def _tc_pre_body(deg_ref, x_ref, w_ref, hp_ref):
    dis, _ = _dis_mask(deg_ref)
    hp_ref[...] = dis * jnp.dot(x_ref[...], w_ref[...],
                                preferred_element_type=jnp.float32)


def _tc_mid_body(p_ref, deg_ref, b_ref, g_ref, beta_ref, w_ref, hp_ref):
    dis, m = _dis_mask(deg_ref)
    t = dis * (p_ref[0] + p_ref[1]) + m * b_ref[...]
    mean = jnp.sum(t, axis=0, keepdims=True) * (1.0 / N)
    var = jnp.sum(t * t, axis=0, keepdims=True) * (1.0 / N) - mean * mean
    y = (t - mean) * lax.rsqrt(var + 1e-5) * g_ref[...] + beta_ref[...]
    y = jnp.maximum(y, 0.0)
    hp_ref[...] = dis * jnp.dot(y, w_ref[...],
                                preferred_element_type=jnp.float32)


def _tc_fin_body(p_ref, deg_ref, b_ref, out_ref):
    dis, m = _dis_mask(deg_ref)
    out_ref[...] = dis * (p_ref[0] + p_ref[1]) + m * b_ref[...]


_hp_shape = jax.ShapeDtypeStruct((NPAD, D), jnp.float32)

_tc_pre = pl.pallas_call(_tc_pre_body, out_shape=_hp_shape)
_tc_mid = pl.pallas_call(_tc_mid_body, out_shape=_hp_shape)
_tc_fin = pl.pallas_call(
    _tc_fin_body, out_shape=jax.ShapeDtypeStruct((NPAD, D), jnp.float32))


def kernel(x, edge_index, W1, b1, g1, beta1, W2, b2, g2, beta2, W3, b3):
    src = edge_index[0]
    dst = edge_index[1]
    padv = N + (jnp.arange(EPAD - E, dtype=jnp.int32) % (NPAD - N))
    src_all = jnp.concatenate([src, padv])
    dst_all = jnp.concatenate([dst, padv])
    dst_p = dst_all.reshape(NW, NCHUNK, CH)
    src0 = src_all[:EA].reshape(NS, CA, CH)
    dst0 = dst_all[:EA].reshape(NS, CA, CH)
    src1 = src_all[EA:].reshape(NS, CB, CH)
    dst1 = dst_all[EA:].reshape(NS, CB, CH)
    x_p = jnp.pad(x, ((0, NPAD - N), (0, 0)))
    zeros_big = jnp.zeros((NPAD, D), jnp.float32)
    ones_rows = jnp.ones((CH, D), jnp.float32)
    b1r = b1.reshape(1, D)
    b2r = b2.reshape(1, D)
    b3r = b3.reshape(1, D)
    g1r = g1.reshape(1, D)
    g2r = g2.reshape(1, D)
    beta1r = beta1.reshape(1, D)
    beta2r = beta2.reshape(1, D)

    _sc_deg, _sc_agg = _sc_kernels()
    deg16 = _sc_deg(dst_p, ones_rows, zeros_big)
    hp1 = _tc_pre(deg16, x_p, W1)
    p1 = _sc_agg(hp1, src0, dst0, src1, dst1, hp1, zeros_big)
    hp2 = _tc_mid(p1, deg16, b1r, g1r, beta1r, W2)
    p2 = _sc_agg(hp2, src0, dst0, src1, dst1, hp2, zeros_big)
    hp3 = _tc_mid(p2, deg16, b2r, g2r, beta2r, W3)
    p3 = _sc_agg(hp3, src0, dst0, src1, dst1, hp3, zeros_big)
    out = _tc_fin(p3, deg16, b3r)
    return out[:N]


# speedup vs baseline: 3.3839x; 1.0036x over previous
import functools

import jax
import jax.numpy as jnp
from jax import lax
from jax.experimental import pallas as pl
from jax.experimental.pallas import tpu as pltpu
from jax.experimental.pallas import tpu_sc as plsc

N = 10000
D = 128
E = 320000
NC = 2
NS = 16
NW = NC * NS
CH = 128
NCHUNK = 80
IB = 8
NB = NCHUNK // IB
EW = CH * NCHUNK
EPAD = EW * NW
NPAD = 10112
RPT = NPAD // NS
NBUF = 2
CA = 80
CB = 80
NBA = CA // IB
NBB = CB // IB
EA = NS * CA * CH


def _sc_deg_body(dst_hbm, ones_hbm, zeros_hbm, out_hbm, didx, ones, acc):
    c = lax.axis_index("c")
    s = lax.axis_index("s")
    w = c * NS + s
    r0 = s * RPT
    pltpu.sync_copy(dst_hbm.at[w], didx)
    pltpu.sync_copy(ones_hbm, ones)
    pltpu.sync_copy(zeros_hbm.at[pl.ds(r0, RPT)], acc.at[pl.ds(r0, RPT)])
    plsc.subcore_barrier()

    def body(j, carry):
        pltpu.sync_copy(ones, acc.at[didx.at[j]], add=True)
        return carry

    lax.fori_loop(0, NCHUNK, body, 0)
    plsc.subcore_barrier()
    pltpu.sync_copy(acc.at[pl.ds(r0, RPT)], out_hbm.at[c, pl.ds(r0, RPT), :])


def _agg_pipeline(hp_hbm, src_hbm, dst_hbm, s, nb, sidx, didx, rows, acc,
                  isem, sems):
    pltpu.sync_copy(src_hbm.at[s, pl.ds(0, IB)], sidx.at[0])
    pltpu.sync_copy(dst_hbm.at[s, pl.ds(0, IB)], didx.at[0])
    for h in range(nb):
        cur = h % 2
        nxt = (h + 1) % 2
        hn = (h + 1) % nb
        pltpu.async_copy(src_hbm.at[s, pl.ds(hn * IB, IB)], sidx.at[nxt], isem)
        pltpu.async_copy(dst_hbm.at[s, pl.ds(hn * IB, IB)], didx.at[nxt], isem)

        for b in range(NBUF):
            pltpu.async_copy(hp_hbm.at[sidx.at[cur, b]], rows.at[b], sems[b])

        def body(g, carry, cur=cur):
            for b in range(NBUF):
                j = g * NBUF + b
                pltpu.make_async_copy(hp_hbm.at[sidx.at[cur, j]], rows.at[b],
                                      sems[b]).wait()
                pltpu.sync_copy(rows.at[b], acc.at[didx.at[cur, j]], add=True)
                jn = lax.rem(j + NBUF, IB)
                pltpu.async_copy(hp_hbm.at[sidx.at[cur, jn]], rows.at[b],
                                 sems[b])
            return carry

        lax.fori_loop(0, IB // NBUF, body, 0)
        for b in range(NBUF):
            pltpu.make_async_copy(hp_hbm.at[sidx.at[cur, b]], rows.at[b],
                                  sems[b]).wait()
        pltpu.make_async_copy(src_hbm.at[s, pl.ds(hn * IB, IB)], sidx.at[nxt],
                              isem).wait()
        pltpu.make_async_copy(dst_hbm.at[s, pl.ds(hn * IB, IB)], didx.at[nxt],
                              isem).wait()


def _sc_agg_body(hp_hbm, src0_hbm, dst0_hbm, src1_hbm, dst1_hbm, init_hbm,
                 zeros_hbm, out_hbm, sidx, didx, rows, acc, isem, s0, s1):
    sems = (s0, s1)
    c = lax.axis_index("c")
    s = lax.axis_index("s")
    r0 = s * RPT

    @pl.when(c == 0)
    def _():
        pltpu.sync_copy(init_hbm.at[pl.ds(r0, RPT)], acc.at[pl.ds(r0, RPT)])

    @pl.when(c != 0)
    def _():
        pltpu.sync_copy(zeros_hbm.at[pl.ds(r0, RPT)], acc.at[pl.ds(r0, RPT)])

    plsc.subcore_barrier()

    @pl.when(c == 0)
    def _():
        _agg_pipeline(hp_hbm, src0_hbm, dst0_hbm, s, NBA, sidx, didx, rows,
                      acc, isem, sems)

    @pl.when(c != 0)
    def _():
        _agg_pipeline(hp_hbm, src1_hbm, dst1_hbm, s, NBB, sidx, didx, rows,
                      acc, isem, sems)

    plsc.subcore_barrier()
    pltpu.sync_copy(acc.at[pl.ds(r0, RPT)], out_hbm.at[c, pl.ds(r0, RPT), :])


@functools.cache
def _sc_kernels():
    mesh = plsc.VectorSubcoreMesh(core_axis_name="c", subcore_axis_name="s",
                                  num_cores=NC, num_subcores=NS)
    sc_deg = pl.kernel(
        _sc_deg_body,
        out_type=jax.ShapeDtypeStruct((NC, NPAD, D), jnp.float32),
        mesh=mesh,
        scratch_types=[
            pltpu.VMEM((NCHUNK, CH), jnp.int32),
            pltpu.VMEM((CH, D), jnp.float32),
            pltpu.VMEM_SHARED((NPAD, D), jnp.float32),
        ],
    )
    sc_agg = pl.kernel(
        _sc_agg_body,
        out_type=jax.ShapeDtypeStruct((NC, NPAD, D), jnp.float32),
        mesh=mesh,
        scratch_types=[
            pltpu.VMEM((2, IB, CH), jnp.int32),
            pltpu.VMEM((2, IB, CH), jnp.int32),
            pltpu.VMEM((NBUF, CH, D), jnp.float32),
            pltpu.VMEM_SHARED((NPAD, D), jnp.float32),
            pltpu.SemaphoreType.DMA,
            pltpu.SemaphoreType.DMA,
            pltpu.SemaphoreType.DMA,
        ],
    )
    return sc_deg, sc_agg


def _dis_mask(deg_ref):
    deg = deg_ref[0, :, 0:1] + deg_ref[1, :, 0:1] + 1.0
    rows = lax.broadcasted_iota(jnp.int32, (NPAD, 1), 0)
    m = (rows < N).astype(jnp.float32)
    return m * lax.rsqrt(deg), m


def _tc_pre_body(deg_ref, x_ref, w_ref, hp_ref):
    dis, _ = _dis_mask(deg_ref)
    hp_ref[...] = dis * jnp.dot(x_ref[...], w_ref[...],
                                preferred_element_type=jnp.float32)


def _tc_mid_body(p_ref, deg_ref, b_ref, g_ref, beta_ref, w_ref, hp_ref):
    dis, m = _dis_mask(deg_ref)
    t = dis * (p_ref[0] + p_ref[1]) + m * b_ref[...]
    mean = jnp.sum(t, axis=0, keepdims=True) * (1.0 / N)
    var = jnp.sum(t * t, axis=0, keepdims=True) * (1.0 / N) - mean * mean
    y = (t - mean) * lax.rsqrt(var + 1e-5) * g_ref[...] + beta_ref[...]
    y = jnp.maximum(y, 0.0)
    hp_ref[...] = dis * jnp.dot(y, w_ref[...],
                                preferred_element_type=jnp.float32)


def _tc_fin_body(p_ref, deg_ref, b_ref, out_ref):
    dis, m = _dis_mask(deg_ref)
    out_ref[...] = dis * (p_ref[0] + p_ref[1]) + m * b_ref[...]


_hp_shape = jax.ShapeDtypeStruct((NPAD, D), jnp.float32)

_tc_pre = pl.pallas_call(_tc_pre_body, out_shape=_hp_shape)
_tc_mid = pl.pallas_call(_tc_mid_body, out_shape=_hp_shape)
_tc_fin = pl.pallas_call(
    _tc_fin_body, out_shape=jax.ShapeDtypeStruct((NPAD, D), jnp.float32))


def kernel(x, edge_index, W1, b1, g1, beta1, W2, b2, g2, beta2, W3, b3):
    src = edge_index[0]
    dst = edge_index[1]
    padv = N + (jnp.arange(EPAD - E, dtype=jnp.int32) % (NPAD - N))
    src_all = jnp.concatenate([src, padv])
    dst_all = jnp.concatenate([dst, padv])
    dst_p = dst_all.reshape(NW, NCHUNK, CH)
    src0 = src_all[:EA].reshape(NS, CA, CH)
    dst0 = dst_all[:EA].reshape(NS, CA, CH)
    src1 = src_all[EA:].reshape(NS, CB, CH)
    dst1 = dst_all[EA:].reshape(NS, CB, CH)
    x_p = jnp.pad(x, ((0, NPAD - N), (0, 0)))
    zeros_big = jnp.zeros((NPAD, D), jnp.float32)
    ones_rows = jnp.ones((CH, D), jnp.float32)
    b1r = b1.reshape(1, D)
    b2r = b2.reshape(1, D)
    b3r = b3.reshape(1, D)
    g1r = g1.reshape(1, D)
    g2r = g2.reshape(1, D)
    beta1r = beta1.reshape(1, D)
    beta2r = beta2.reshape(1, D)

    _sc_deg, _sc_agg = _sc_kernels()
    deg16 = _sc_deg(dst_p, ones_rows, zeros_big)
    hp1 = _tc_pre(deg16, x_p, W1)
    p1 = _sc_agg(hp1, src0, dst0, src1, dst1, hp1, zeros_big)
    hp2 = _tc_mid(p1, deg16, b1r, g1r, beta1r, W2)
    p2 = _sc_agg(hp2, src0, dst0, src1, dst1, hp2, zeros_big)
    hp3 = _tc_mid(p2, deg16, b2r, g2r, beta2r, W3)
    p3 = _sc_agg(hp3, src0, dst0, src1, dst1, hp3, zeros_big)
    out = _tc_fin(p3, deg16, b3r)
    return out[:N]
